# baseline (device time: 428115 ns/iter reference)
import jax
import jax.numpy as jnp
from jax import lax
from jax.experimental import pallas as pl
from jax.experimental.pallas import tpu as pltpu

W = 32


def kernel(x, w_mat, scale_x, scale_w):
    m, k_sh = x.shape
    _, n = w_mat.shape
    mc = m // W
    nh = n // 2

    def body(x_ref, w_ref, sx_ref, sw_ref, out_ref,
             acc_cw, recv_cw, acc_ccw, recv_ccw,
             send_sems_cw, recv_sems_cw, send_sems_ccw, recv_sems_ccw,
             credit_cw, credit_ccw):
        my = lax.axis_index("i")
        left = lax.rem(my + W - 1, W)
        right = lax.rem(my + 1, W)

        barrier = pltpu.get_barrier_semaphore()
        for nbr in (left, right):
            pl.semaphore_signal(
                barrier, inc=1,
                device_id=(nbr,), device_id_type=pl.DeviceIdType.MESH,
            )
        pl.semaphore_wait(barrier, 2)

        def pchunk(c, half):
            xc = x_ref[pl.ds(c * mc, mc), :]
            wc = w_ref[:, half * nh:(half + 1) * nh]
            return lax.dot_general(
                xc, wc,
                dimension_numbers=(((1,), (0,)), ((), ())),
                preferred_element_type=jnp.int32,
            )

        def make_rdma(slot, acc, recv, ssems, rsems, dst):
            return pltpu.make_async_remote_copy(
                src_ref=acc.at[slot],
                dst_ref=recv.at[slot],
                send_sem=ssems.at[slot],
                recv_sem=rsems.at[slot],
                device_id=(dst,),
                device_id_type=pl.DeviceIdType.MESH,
            )

        def mk_cw(slot):
            return make_rdma(slot, acc_cw, recv_cw,
                             send_sems_cw, recv_sems_cw, right)

        def mk_ccw(slot):
            return make_rdma(slot, acc_ccw, recv_ccw,
                             send_sems_ccw, recv_sems_ccw, left)

        acc_cw[0] = pchunk(lax.rem(my + W - 1, W), 0)
        acc_ccw[0] = pchunk(lax.rem(my + 1, W), 1)
        rd_cw = [mk_cw(0)]
        rd_ccw = [mk_ccw(0)]
        rd_cw[0].start()
        rd_ccw[0].start()

        for h in range(W - 1):
            c_cw = lax.rem(my + 2 * W - 2 - h, W)
            c_ccw = lax.rem(my + 2 + h, W)
            p_cw = pchunk(c_cw, 0)
            p_ccw = pchunk(c_ccw, 1)
            rd_cw[h].wait_recv()
            total_cw = recv_cw[h % 2] + p_cw
            rd_ccw[h].wait_recv()
            total_ccw = recv_ccw[h % 2] + p_ccw
            if h < W - 2:
                if h >= 1:
                    rd_cw[h - 1].wait_send()
                    rd_ccw[h - 1].wait_send()
                acc_cw[(h + 1) % 2] = total_cw
                acc_ccw[(h + 1) % 2] = total_ccw
                if h >= 1:
                    pl.semaphore_wait(credit_cw, 1)
                    pl.semaphore_wait(credit_ccw, 1)
                r = mk_cw((h + 1) % 2)
                r.start()
                rd_cw.append(r)
                r = mk_ccw((h + 1) % 2)
                r.start()
                rd_ccw.append(r)
            else:
                scale = sx_ref[0] * sw_ref[0]
                out_ref[:, 0:nh] = total_cw.astype(jnp.float32) * scale
                out_ref[:, nh:n] = total_ccw.astype(jnp.float32) * scale
            if h <= W - 4:
                pl.semaphore_signal(
                    credit_cw, inc=1,
                    device_id=(left,), device_id_type=pl.DeviceIdType.MESH,
                )
                pl.semaphore_signal(
                    credit_ccw, inc=1,
                    device_id=(right,), device_id_type=pl.DeviceIdType.MESH,
                )

        for rd in (rd_cw, rd_ccw):
            rd[W - 3].wait_send()
            rd[W - 2].wait_send()

    return pl.pallas_call(
        body,
        out_shape=jax.ShapeDtypeStruct((mc, n), jnp.float32),
        in_specs=[
            pl.BlockSpec(memory_space=pltpu.VMEM),
            pl.BlockSpec(memory_space=pltpu.VMEM),
            pl.BlockSpec(memory_space=pltpu.SMEM),
            pl.BlockSpec(memory_space=pltpu.SMEM),
        ],
        out_specs=pl.BlockSpec(memory_space=pltpu.VMEM),
        scratch_shapes=[
            pltpu.VMEM((2, mc, nh), jnp.int32),
            pltpu.VMEM((2, mc, nh), jnp.int32),
            pltpu.VMEM((2, mc, nh), jnp.int32),
            pltpu.VMEM((2, mc, nh), jnp.int32),
            pltpu.SemaphoreType.DMA((2,)),
            pltpu.SemaphoreType.DMA((2,)),
            pltpu.SemaphoreType.DMA((2,)),
            pltpu.SemaphoreType.DMA((2,)),
            pltpu.SemaphoreType.REGULAR,
            pltpu.SemaphoreType.REGULAR,
        ],
        compiler_params=pltpu.CompilerParams(collective_id=0),
    )(x, w_mat, scale_x, scale_w)


# device time: 242931 ns/iter; 1.7623x vs baseline; 1.7623x over previous
import jax
import jax.numpy as jnp
from jax import lax
from jax.experimental import pallas as pl
from jax.experimental.pallas import tpu as pltpu

W = 32

PI = [0, 8, 16, 24, 27, 19, 11, 12, 20, 28, 31, 23, 15, 7, 4, 3,
      2, 5, 6, 14, 22, 30, 29, 21, 13, 10, 18, 26, 25, 17, 9, 1]
PI_INV = [0] * W
for _r, _m in enumerate(PI):
    PI_INV[_m] = _r


def kernel(x, w_mat, scale_x, scale_w):
    m, k_sh = x.shape
    _, n = w_mat.shape
    mc = m // W
    nh = n // 2

    pi = jnp.asarray(PI, dtype=jnp.int32)
    pi_inv = jnp.asarray(PI_INV, dtype=jnp.int32)
    my = lax.axis_index("i")
    r = pi_inv[my]
    s_idx = jnp.arange(W, dtype=jnp.int32)
    cs_cw = pi[(r - 1 - s_idx) % W]
    cs_ccw = pi[(r + 1 + s_idx) % W]
    nbrs = jnp.stack([pi[(r - 1) % W], pi[(r + 1) % W]])

    def body(nbr_ref, cs_cw_ref, cs_ccw_ref, x_ref, w_ref, sx_ref, sw_ref,
             out_ref,
             acc_cw, recv_cw, acc_ccw, recv_ccw,
             send_sems_cw, recv_sems_cw, send_sems_ccw, recv_sems_ccw,
             credit_cw, credit_ccw):
        left = nbr_ref[0]
        right = nbr_ref[1]

        barrier = pltpu.get_barrier_semaphore()
        for nbr in (left, right):
            pl.semaphore_signal(
                barrier, inc=1,
                device_id=(nbr,), device_id_type=pl.DeviceIdType.MESH,
            )
        pl.semaphore_wait(barrier, 2)

        def pchunk(c, half):
            xc = x_ref[pl.ds(c * mc, mc), :]
            wc = w_ref[:, half * nh:(half + 1) * nh]
            return lax.dot_general(
                xc, wc,
                dimension_numbers=(((1,), (0,)), ((), ())),
                preferred_element_type=jnp.int32,
            )

        def make_rdma(slot, acc, recv, ssems, rsems, dst):
            return pltpu.make_async_remote_copy(
                src_ref=acc.at[slot],
                dst_ref=recv.at[slot],
                send_sem=ssems.at[slot],
                recv_sem=rsems.at[slot],
                device_id=(dst,),
                device_id_type=pl.DeviceIdType.MESH,
            )

        def mk_cw(slot):
            return make_rdma(slot, acc_cw, recv_cw,
                             send_sems_cw, recv_sems_cw, right)

        def mk_ccw(slot):
            return make_rdma(slot, acc_ccw, recv_ccw,
                             send_sems_ccw, recv_sems_ccw, left)

        acc_cw[0] = pchunk(cs_cw_ref[0], 0)
        acc_ccw[0] = pchunk(cs_ccw_ref[0], 1)
        rd_cw = [mk_cw(0)]
        rd_ccw = [mk_ccw(0)]
        rd_cw[0].start()
        rd_ccw[0].start()

        for h in range(W - 1):
            p_cw = pchunk(cs_cw_ref[h + 1], 0)
            p_ccw = pchunk(cs_ccw_ref[h + 1], 1)
            rd_cw[h].wait_recv()
            total_cw = recv_cw[h % 2] + p_cw
            rd_ccw[h].wait_recv()
            total_ccw = recv_ccw[h % 2] + p_ccw
            if h < W - 2:
                if h >= 1:
                    rd_cw[h - 1].wait_send()
                    rd_ccw[h - 1].wait_send()
                acc_cw[(h + 1) % 2] = total_cw
                acc_ccw[(h + 1) % 2] = total_ccw
                if h >= 1:
                    pl.semaphore_wait(credit_cw, 1)
                    pl.semaphore_wait(credit_ccw, 1)
                rd = mk_cw((h + 1) % 2)
                rd.start()
                rd_cw.append(rd)
                rd = mk_ccw((h + 1) % 2)
                rd.start()
                rd_ccw.append(rd)
            else:
                scale = sx_ref[0] * sw_ref[0]
                out_ref[:, 0:nh] = total_cw.astype(jnp.float32) * scale
                out_ref[:, nh:n] = total_ccw.astype(jnp.float32) * scale
            if h <= W - 4:
                pl.semaphore_signal(
                    credit_cw, inc=1,
                    device_id=(left,), device_id_type=pl.DeviceIdType.MESH,
                )
                pl.semaphore_signal(
                    credit_ccw, inc=1,
                    device_id=(right,), device_id_type=pl.DeviceIdType.MESH,
                )

        for rd in (rd_cw, rd_ccw):
            rd[W - 3].wait_send()
            rd[W - 2].wait_send()

    return pl.pallas_call(
        body,
        out_shape=jax.ShapeDtypeStruct((mc, n), jnp.float32),
        in_specs=[
            pl.BlockSpec(memory_space=pltpu.SMEM),
            pl.BlockSpec(memory_space=pltpu.SMEM),
            pl.BlockSpec(memory_space=pltpu.SMEM),
            pl.BlockSpec(memory_space=pltpu.VMEM),
            pl.BlockSpec(memory_space=pltpu.VMEM),
            pl.BlockSpec(memory_space=pltpu.SMEM),
            pl.BlockSpec(memory_space=pltpu.SMEM),
        ],
        out_specs=pl.BlockSpec(memory_space=pltpu.VMEM),
        scratch_shapes=[
            pltpu.VMEM((2, mc, nh), jnp.int32),
            pltpu.VMEM((2, mc, nh), jnp.int32),
            pltpu.VMEM((2, mc, nh), jnp.int32),
            pltpu.VMEM((2, mc, nh), jnp.int32),
            pltpu.SemaphoreType.DMA((2,)),
            pltpu.SemaphoreType.DMA((2,)),
            pltpu.SemaphoreType.DMA((2,)),
            pltpu.SemaphoreType.DMA((2,)),
            pltpu.SemaphoreType.REGULAR,
            pltpu.SemaphoreType.REGULAR,
        ],
        compiler_params=pltpu.CompilerParams(collective_id=0),
    )(nbrs, cs_cw, cs_ccw, x, w_mat, scale_x, scale_w)


# device time: 185811 ns/iter; 2.3040x vs baseline; 1.3074x over previous
import jax
import jax.numpy as jnp
from jax import lax
from jax.experimental import pallas as pl
from jax.experimental.pallas import tpu as pltpu

W = 32

PI = [0, 8, 16, 24, 27, 19, 11, 12, 20, 28, 31, 23, 15, 7, 4, 3,
      2, 5, 6, 14, 22, 30, 29, 21, 13, 10, 18, 26, 25, 17, 9, 1]
PI_INV = [0] * W
for _r, _m in enumerate(PI):
    PI_INV[_m] = _r


def kernel(x, w_mat, scale_x, scale_w):
    m, k_sh = x.shape
    _, n = w_mat.shape
    mc = m // W
    nh = n // 2
    nq = nh // 2

    pi = jnp.asarray(PI, dtype=jnp.int32)
    pi_inv = jnp.asarray(PI_INV, dtype=jnp.int32)
    my = lax.axis_index("i")
    r = pi_inv[my]
    s_idx = jnp.arange(W, dtype=jnp.int32)
    cs_cw = pi[(r - 1 - s_idx) % W]
    cs_ccw = pi[(r + 1 + s_idx) % W]
    nbrs = jnp.stack([pi[(r - 1) % W], pi[(r + 1) % W]])

    def body(nbr_ref, cs_cw_ref, cs_ccw_ref, x_ref, w_ref, sx_ref, sw_ref,
             out_ref,
             acc_cw, recv_cw, acc_ccw, recv_ccw,
             send_sems_cw, recv_sems_cw, send_sems_ccw, recv_sems_ccw,
             credit_cw, credit_ccw):
        left = nbr_ref[0]
        right = nbr_ref[1]

        barrier = pltpu.get_barrier_semaphore()
        for nbr in (left, right):
            pl.semaphore_signal(
                barrier, inc=1,
                device_id=(nbr,), device_id_type=pl.DeviceIdType.MESH,
            )
        pl.semaphore_wait(barrier, 2)

        def pchunk(c, half):
            xc = x_ref[pl.ds(c * mc, mc), :]
            wc = w_ref[:, half * nh:(half + 1) * nh]
            return lax.dot_general(
                xc, wc,
                dimension_numbers=(((1,), (0,)), ((), ())),
                preferred_element_type=jnp.int32,
            )

        def make_rdma(slot, f, acc, recv, ssems, rsems, dst):
            lo = f * nq
            return pltpu.make_async_remote_copy(
                src_ref=acc.at[slot, :, lo:lo + nq],
                dst_ref=recv.at[slot, :, lo:lo + nq],
                send_sem=ssems.at[slot, f],
                recv_sem=rsems.at[slot, f],
                device_id=(dst,),
                device_id_type=pl.DeviceIdType.MESH,
            )

        def mk_cw(slot, f):
            return make_rdma(slot, f, acc_cw, recv_cw,
                             send_sems_cw, recv_sems_cw, right)

        def mk_ccw(slot, f):
            return make_rdma(slot, f, acc_ccw, recv_ccw,
                             send_sems_ccw, recv_sems_ccw, left)

        scale = sx_ref[0] * sw_ref[0]

        acc_cw[0] = pchunk(cs_cw_ref[0], 0)
        acc_ccw[0] = pchunk(cs_ccw_ref[0], 1)
        rd_cw = [[mk_cw(0, 0), mk_cw(0, 1)]]
        rd_ccw = [[mk_ccw(0, 0), mk_ccw(0, 1)]]
        rd_cw[0][0].start()
        rd_ccw[0][0].start()
        rd_cw[0][1].start()
        rd_ccw[0][1].start()

        def do_flit(h, f, rds, acc, recv, p, credit, mk, out_lo):
            rds[h][f].wait_recv()
            lo = f * nq
            tot = recv[h % 2, :, lo:lo + nq] + p[:, lo:lo + nq]
            if h < W - 2:
                if h >= 1:
                    rds[h - 1][f].wait_send()
                acc[(h + 1) % 2, :, lo:lo + nq] = tot
                if h >= 1 and f == 0:
                    pl.semaphore_wait(credit, 1)
                rd = mk((h + 1) % 2, f)
                rd.start()
                return rd
            out_ref[:, out_lo + lo:out_lo + lo + nq] = (
                tot.astype(jnp.float32) * scale
            )
            return None

        for h in range(W - 1):
            p_cw = pchunk(cs_cw_ref[h + 1], 0)
            p_ccw = pchunk(cs_ccw_ref[h + 1], 1)
            a_cw = do_flit(h, 0, rd_cw, acc_cw, recv_cw, p_cw,
                           credit_cw, mk_cw, 0)
            a_ccw = do_flit(h, 0, rd_ccw, acc_ccw, recv_ccw, p_ccw,
                            credit_ccw, mk_ccw, nh)
            b_cw = do_flit(h, 1, rd_cw, acc_cw, recv_cw, p_cw,
                           credit_cw, mk_cw, 0)
            b_ccw = do_flit(h, 1, rd_ccw, acc_ccw, recv_ccw, p_ccw,
                            credit_ccw, mk_ccw, nh)
            rd_cw.append([a_cw, b_cw])
            rd_ccw.append([a_ccw, b_ccw])
            if h <= W - 4:
                pl.semaphore_signal(
                    credit_cw, inc=1,
                    device_id=(left,), device_id_type=pl.DeviceIdType.MESH,
                )
                pl.semaphore_signal(
                    credit_ccw, inc=1,
                    device_id=(right,), device_id_type=pl.DeviceIdType.MESH,
                )

        for rds in (rd_cw, rd_ccw):
            for hh in (W - 3, W - 2):
                rds[hh][0].wait_send()
                rds[hh][1].wait_send()

    return pl.pallas_call(
        body,
        out_shape=jax.ShapeDtypeStruct((mc, n), jnp.float32),
        in_specs=[
            pl.BlockSpec(memory_space=pltpu.SMEM),
            pl.BlockSpec(memory_space=pltpu.SMEM),
            pl.BlockSpec(memory_space=pltpu.SMEM),
            pl.BlockSpec(memory_space=pltpu.VMEM),
            pl.BlockSpec(memory_space=pltpu.VMEM),
            pl.BlockSpec(memory_space=pltpu.SMEM),
            pl.BlockSpec(memory_space=pltpu.SMEM),
        ],
        out_specs=pl.BlockSpec(memory_space=pltpu.VMEM),
        scratch_shapes=[
            pltpu.VMEM((2, mc, nh), jnp.int32),
            pltpu.VMEM((2, mc, nh), jnp.int32),
            pltpu.VMEM((2, mc, nh), jnp.int32),
            pltpu.VMEM((2, mc, nh), jnp.int32),
            pltpu.SemaphoreType.DMA((2, 2)),
            pltpu.SemaphoreType.DMA((2, 2)),
            pltpu.SemaphoreType.DMA((2, 2)),
            pltpu.SemaphoreType.DMA((2, 2)),
            pltpu.SemaphoreType.REGULAR,
            pltpu.SemaphoreType.REGULAR,
        ],
        compiler_params=pltpu.CompilerParams(collective_id=0),
    )(nbrs, cs_cw, cs_ccw, x, w_mat, scale_x, scale_w)
